# baseline (speedup 1.0000x reference)
import functools
import jax
import jax.numpy as jnp
from jax import lax
from jax.experimental import pallas as pl
from jax.experimental.pallas import tpu as pltpu
from jax.experimental.pallas import tpu_sc as plsc

NC = 2
NS = 16
NW = NC * NS
L = 16


def kernel(x, pos_table):
    B, S, D = x.shape
    xf = x.reshape(B * S * D)
    pf = pos_table.reshape(-1)
    rows_per_w = S // NW
    CH = 32
    n_ch = rows_per_w // CH
    CHW = CH * D
    U = 8

    mesh = plsc.VectorSubcoreMesh(
        core_axis_name="c", subcore_axis_name="s", num_cores=NC, num_subcores=NS
    )

    @functools.partial(
        pl.kernel,
        mesh=mesh,
        out_type=jax.ShapeDtypeStruct((B * S * D,), jnp.float32),
        scratch_types=[
            pltpu.VMEM((CHW,), jnp.float32),
            pltpu.VMEM((CHW,), jnp.float32),
        ],
    )
    def body(x_hbm, pos_hbm, out_hbm, posb, xb):
        wid = lax.axis_index("s") * NC + lax.axis_index("c")
        base_seq = wid * rows_per_w
        for c in range(n_ch):
            pos_off = (base_seq + c * CH) * D
            pltpu.sync_copy(pos_hbm.at[pl.ds(pos_off, CHW)], posb)
            for b in range(B):
                x_off = (b * S) * D + pos_off
                pltpu.sync_copy(x_hbm.at[pl.ds(x_off, CHW)], xb)

                @plsc.parallel_loop(0, CHW, step=L, unroll=U)
                def add_body(i):
                    plsc.addupdate(xb.at[pl.ds(i, L)], posb[pl.ds(i, L)])

                pltpu.sync_copy(xb, out_hbm.at[pl.ds(x_off, CHW)])

    out = body(xf, pf)
    return out.reshape(B, S, D)
